# baseline (speedup 1.0000x reference)
import jax
import jax.numpy as jnp
from jax.experimental import pallas as pl
from jax.experimental.pallas import tpu as pltpu

L, C, N, M, BITS = 100, 100, 13000, 200000, 64
W = 12800
NSTEPS = 15
REM = M - W * NSTEPS


def _count(aBb, db, width):
    matc = jax.lax.dot_general(aBb, db, (((1,), (0,)), ((), ())),
                               preferred_element_type=jnp.float32)
    eq = (matc > float(BITS - 2)).astype(jnp.bfloat16)
    ones = jnp.ones((width, 128), jnp.bfloat16)
    return jax.lax.dot_general(eq, ones, (((1,), (0,)), ((), ())),
                               preferred_element_type=jnp.float32)


def _body(tl_ref, trlT_ref, tcT_ref, rem_ref, dbT_ref, aB_ref, avg_ref,
          zero_ref, aBb_ref, buf_ref, trl_buf, tc_buf, rem_buf, freq_ref,
          sem_ref, isem_ref):
    j = pl.program_id(0)

    def start_copy(slot, blk):
        pltpu.make_async_copy(
            dbT_ref.at[:, pl.ds(pl.multiple_of(blk * W, 128), W)],
            buf_ref.at[slot],
            sem_ref.at[slot],
        ).start()

    @pl.when(j == 0)
    def _init():
        start_copy(0, 0)
        start_copy(1, 1)
        cp_trl = pltpu.make_async_copy(trlT_ref, trl_buf, isem_ref.at[0])
        cp_tc = pltpu.make_async_copy(tcT_ref, tc_buf, isem_ref.at[1])
        cp_rem = pltpu.make_async_copy(rem_ref, rem_buf, isem_ref.at[2])
        cp_trl.start()
        cp_tc.start()
        cp_rem.start()
        cp_trl.wait()
        cp_tc.wait()
        tl = tl_ref[...].astype(jnp.bfloat16)
        trlT = trl_buf[...].astype(jnp.bfloat16)
        simd = jax.lax.dot_general(tl, trlT, (((1,), (0,)), ((), ())),
                                   preferred_element_type=jnp.float32)
        sim = (simd > 0.0).astype(jnp.bfloat16)
        svote = jax.lax.dot_general(sim, tc_buf[...].astype(jnp.bfloat16),
                                    (((1,), (1,)), ((), ())),
                                    preferred_element_type=jnp.float32)
        aB = jnp.sign(svote)
        aB_ref[...] = aB
        aBb = aB.astype(jnp.bfloat16)
        aBb_ref[...] = aBb
        cp_rem.wait()
        freq_ref[...] = _count(aBb, rem_buf[...].astype(jnp.bfloat16), REM)

    slot = jax.lax.rem(j, 2)
    pltpu.make_async_copy(
        dbT_ref.at[:, pl.ds(pl.multiple_of(j * W, 128), W)],
        buf_ref.at[slot],
        sem_ref.at[slot],
    ).wait()
    db = buf_ref[slot].astype(jnp.bfloat16)
    freq_ref[...] += _count(aBb_ref[...], db, W)

    @pl.when(j + 2 < NSTEPS)
    def _next():
        start_copy(slot, j + 2)

    @pl.when(j == NSTEPS - 1)
    def _fini():
        freq = freq_ref[...][:, 0:1]
        avg_ref[...] = (jnp.sum(freq) / float(L)).reshape(1, 1)
        zero_ref[...] = jnp.sum((freq == 0.0).astype(jnp.float32)).reshape(1, 1)


def kernel(traincodes, dB, target_labels, trainlabels):
    trlT = trainlabels.T
    tcT = traincodes.T
    dBT = dB.T
    rem = dBT[:, W * NSTEPS:]

    aB, avg, zero = pl.pallas_call(
        _body,
        grid=(NSTEPS,),
        in_specs=[
            pl.BlockSpec((L, C), lambda j: (0, 0)),
            pl.BlockSpec(memory_space=pl.ANY),
            pl.BlockSpec(memory_space=pl.ANY),
            pl.BlockSpec(memory_space=pl.ANY),
            pl.BlockSpec(memory_space=pl.ANY),
        ],
        out_specs=[
            pl.BlockSpec((L, BITS), lambda j: (0, 0)),
            pl.BlockSpec((1, 1), lambda j: (0, 0)),
            pl.BlockSpec((1, 1), lambda j: (0, 0)),
        ],
        out_shape=[
            jax.ShapeDtypeStruct((L, BITS), jnp.float32),
            jax.ShapeDtypeStruct((1, 1), jnp.float32),
            jax.ShapeDtypeStruct((1, 1), jnp.float32),
        ],
        scratch_shapes=[
            pltpu.VMEM((L, BITS), jnp.bfloat16),
            pltpu.VMEM((2, BITS, W), jnp.float32),
            pltpu.VMEM((C, N), jnp.int32),
            pltpu.VMEM((BITS, N), jnp.float32),
            pltpu.VMEM((BITS, REM), jnp.float32),
            pltpu.VMEM((L, 128), jnp.float32),
            pltpu.SemaphoreType.DMA((2,)),
            pltpu.SemaphoreType.DMA((3,)),
        ],
    )(target_labels, trlT, tcT, rem, dBT)
    return (aB, avg[0, 0], zero[0, 0])


# speedup vs baseline: 1.0356x; 1.0356x over previous
import jax
import jax.numpy as jnp
from jax.experimental import pallas as pl
from jax.experimental.pallas import tpu as pltpu

L, C, N, M, BITS = 100, 100, 13000, 200000, 64
W = 12800
NSTEPS = 15
REM = M - W * NSTEPS


def _count(aBb, db, width):
    matc = jax.lax.dot_general(aBb, db, (((1,), (0,)), ((), ())),
                               preferred_element_type=jnp.float32)
    eq = (matc > float(BITS - 2)).astype(jnp.bfloat16)
    ones = jnp.ones((width, 128), jnp.bfloat16)
    return jax.lax.dot_general(eq, ones, (((1,), (0,)), ((), ())),
                               preferred_element_type=jnp.float32)


def _body(tl_ref, trlT_ref, tcT_ref, rem_ref, dbT_ref, aB_ref, avg_ref,
          zero_ref, aBb_ref, buf_ref, freq_ref, sem_ref):
    j = pl.program_id(0)

    def start_copy(slot, blk):
        off = pl.multiple_of(blk * W, 128)
        pltpu.make_async_copy(
            dbT_ref.at[0:32, pl.ds(off, W)],
            buf_ref.at[slot, 0:32],
            sem_ref.at[slot, 0],
        ).start()
        pltpu.make_async_copy(
            dbT_ref.at[32:64, pl.ds(off, W)],
            buf_ref.at[slot, 32:64],
            sem_ref.at[slot, 1],
        ).start()

    @pl.when(j == 0)
    def _init():
        start_copy(0, 0)
        start_copy(1, 1)
        tl = tl_ref[...].astype(jnp.bfloat16)
        trlT = trlT_ref[...].astype(jnp.bfloat16)
        simd = jax.lax.dot_general(tl, trlT, (((1,), (0,)), ((), ())),
                                   preferred_element_type=jnp.float32)
        sim = (simd > 0.0).astype(jnp.bfloat16)
        svote = jax.lax.dot_general(sim, tcT_ref[...].astype(jnp.bfloat16),
                                    (((1,), (1,)), ((), ())),
                                    preferred_element_type=jnp.float32)
        aB = jnp.sign(svote)
        aB_ref[...] = aB
        aBb = aB.astype(jnp.bfloat16)
        aBb_ref[...] = aBb
        freq_ref[...] = _count(aBb, rem_ref[...].astype(jnp.bfloat16), REM)

    slot = jax.lax.rem(j, 2)
    off = pl.multiple_of(j * W, 128)
    pltpu.make_async_copy(
        dbT_ref.at[0:32, pl.ds(off, W)],
        buf_ref.at[slot, 0:32],
        sem_ref.at[slot, 0],
    ).wait()
    pltpu.make_async_copy(
        dbT_ref.at[32:64, pl.ds(off, W)],
        buf_ref.at[slot, 32:64],
        sem_ref.at[slot, 1],
    ).wait()
    db = buf_ref[slot].astype(jnp.bfloat16)
    freq_ref[...] += _count(aBb_ref[...], db, W)

    @pl.when(j + 2 < NSTEPS)
    def _next():
        start_copy(slot, j + 2)

    @pl.when(j == NSTEPS - 1)
    def _fini():
        freq = freq_ref[...][:, 0:1]
        avg_ref[...] = (jnp.sum(freq) / float(L)).reshape(1, 1)
        zero_ref[...] = jnp.sum((freq == 0.0).astype(jnp.float32)).reshape(1, 1)


def kernel(traincodes, dB, target_labels, trainlabels):
    trlT = trainlabels.T
    tcT = traincodes.T
    dBT = dB.T
    rem = dBT[:, W * NSTEPS:]

    aB, avg, zero = pl.pallas_call(
        _body,
        grid=(NSTEPS,),
        in_specs=[
            pl.BlockSpec((L, C), lambda j: (0, 0)),
            pl.BlockSpec((C, N), lambda j: (0, 0)),
            pl.BlockSpec((BITS, N), lambda j: (0, 0)),
            pl.BlockSpec((BITS, REM), lambda j: (0, 0)),
            pl.BlockSpec(memory_space=pl.ANY),
        ],
        out_specs=[
            pl.BlockSpec((L, BITS), lambda j: (0, 0)),
            pl.BlockSpec((1, 1), lambda j: (0, 0)),
            pl.BlockSpec((1, 1), lambda j: (0, 0)),
        ],
        out_shape=[
            jax.ShapeDtypeStruct((L, BITS), jnp.float32),
            jax.ShapeDtypeStruct((1, 1), jnp.float32),
            jax.ShapeDtypeStruct((1, 1), jnp.float32),
        ],
        scratch_shapes=[
            pltpu.VMEM((L, BITS), jnp.bfloat16),
            pltpu.VMEM((2, BITS, W), jnp.float32),
            pltpu.VMEM((L, 128), jnp.float32),
            pltpu.SemaphoreType.DMA((2, 2)),
        ],
    )(target_labels, trlT, tcT, rem, dBT)
    return (aB, avg[0, 0], zero[0, 0])
